# Initial kernel scaffold; baseline (speedup 1.0000x reference)
import jax
import jax.numpy as jnp
from jax.experimental import pallas as pl


def kernel(category, rotation_z, location, dimension, edge_feat, W_cat, W_orient, W_size, b_size, W_trans, b_trans, W_node, b_node, W_edge):
    raise NotImplementedError("write your pallas kernel here")


# speedup vs baseline: 3.9448x; 3.9448x over previous
import jax
import jax.numpy as jnp
from jax import lax
from jax.experimental import pallas as pl

N_NODES = 50000
N_EDGES = 800000
E_BLK = 6400
N_BLK = 2000
CAT_P = 1024
OR_P = 384
ED_P = 32


def _edge_body(idx_ref, tbl_ref, out_ref):
    idx = idx_ref[0, 0, :]
    tbl = jnp.maximum(tbl_ref[...], 0.0)
    oh = (lax.broadcasted_iota(jnp.int32, (E_BLK, ED_P), 1)
          == idx[:, None]).astype(jnp.float32)
    out_ref[...] = jnp.dot(oh, tbl, preferred_element_type=jnp.float32)


def _node_body(cat_ref, rot_ref, dl_ref, Wc_ref, Wo_ref, Wn1_ref, Wn2_ref,
               Wn3_ref, Wn4_ref, Ws_ref, Wt_ref, bs_ref, bt_ref, bn_ref,
               out_ref):
    Tcat = lax.dot_general(Wc_ref[...], Wn1_ref[...],
                           (((1,), (1,)), ((), ())),
                           preferred_element_type=jnp.float32)
    Tor = lax.dot_general(Wo_ref[...], Wn2_ref[...],
                          (((1,), (1,)), ((), ())),
                          preferred_element_type=jnp.float32)
    A_dim = lax.dot_general(Ws_ref[...], Wn3_ref[...],
                            (((0,), (1,)), ((), ())),
                            preferred_element_type=jnp.float32)
    A_loc = lax.dot_general(Wt_ref[...], Wn4_ref[...],
                            (((0,), (1,)), ((), ())),
                            preferred_element_type=jnp.float32)
    b_eff = (lax.dot_general(bs_ref[...], Wn3_ref[...],
                             (((1,), (1,)), ((), ())),
                             preferred_element_type=jnp.float32)
             + lax.dot_general(bt_ref[...], Wn4_ref[...],
                               (((1,), (1,)), ((), ())),
                               preferred_element_type=jnp.float32)
             + bn_ref[...])
    A = jnp.concatenate([A_dim, A_loc, b_eff, jnp.zeros((1, 64),
                                                        jnp.float32)], axis=0)

    cat = cat_ref[0, 0, :]
    rot = rot_ref[0, 0, :]
    ohc = (lax.broadcasted_iota(jnp.int32, (N_BLK, CAT_P), 1)
           == cat[:, None]).astype(jnp.float32)
    oho = (lax.broadcasted_iota(jnp.int32, (N_BLK, OR_P), 1)
           == rot[:, None]).astype(jnp.float32)
    acc = jnp.dot(ohc, Tcat, preferred_element_type=jnp.float32)
    acc += jnp.dot(oho, Tor, preferred_element_type=jnp.float32)
    acc += jnp.dot(dl_ref[...], A, preferred_element_type=jnp.float32)
    out_ref[...] = jnp.maximum(acc, 0.0)


def kernel(category, rotation_z, location, dimension, edge_feat,
           W_cat, W_orient, W_size, b_size, W_trans, b_trans,
           W_node, b_node, W_edge):
    f32 = jnp.float32
    cat3 = category.astype(jnp.int32).reshape(N_NODES // N_BLK, 1, N_BLK)
    rot3 = rotation_z.astype(jnp.int32).reshape(N_NODES // N_BLK, 1, N_BLK)
    eidx = edge_feat.astype(jnp.int32).reshape(N_EDGES // E_BLK, 1, E_BLK)
    dl = jnp.concatenate(
        [dimension.astype(f32), location.astype(f32),
         jnp.ones((N_NODES, 1), f32), jnp.zeros((N_NODES, 1), f32)], axis=1)
    Wc_p = jnp.pad(W_cat, ((0, CAT_P - W_cat.shape[0]), (0, 0)))
    Wo_p = jnp.pad(W_orient, ((0, OR_P - W_orient.shape[0]), (0, 0)))
    We_p = jnp.pad(W_edge, ((0, ED_P - W_edge.shape[0]), (0, 0)))
    Wn1 = W_node[:, 0:64]
    Wn2 = W_node[:, 64:96]
    Wn3 = W_node[:, 96:112]
    Wn4 = W_node[:, 112:128]
    bs2 = b_size.reshape(1, 16)
    bt2 = b_trans.reshape(1, 16)
    bn2 = b_node.reshape(1, 64)

    m_edge = pl.pallas_call(
        _edge_body,
        grid=(N_EDGES // E_BLK,),
        in_specs=[
            pl.BlockSpec((1, 1, E_BLK), lambda i: (i, 0, 0)),
            pl.BlockSpec((ED_P, 64), lambda i: (0, 0)),
        ],
        out_specs=pl.BlockSpec((E_BLK, 64), lambda i: (i, 0)),
        out_shape=jax.ShapeDtypeStruct((N_EDGES, 64), f32),
    )(eidx, We_p)

    full = lambda shape: pl.BlockSpec(shape, lambda i: tuple(0 for _ in shape))
    m_node = pl.pallas_call(
        _node_body,
        grid=(N_NODES // N_BLK,),
        in_specs=[
            pl.BlockSpec((1, 1, N_BLK), lambda i: (i, 0, 0)),
            pl.BlockSpec((1, 1, N_BLK), lambda i: (i, 0, 0)),
            pl.BlockSpec((N_BLK, 8), lambda i: (i, 0)),
            full((CAT_P, 64)), full((OR_P, 32)),
            full((64, 64)), full((64, 32)), full((64, 16)), full((64, 16)),
            full((16, 3)), full((16, 3)),
            full((1, 16)), full((1, 16)), full((1, 64)),
        ],
        out_specs=pl.BlockSpec((N_BLK, 64), lambda i: (i, 0)),
        out_shape=jax.ShapeDtypeStruct((N_NODES, 64), f32),
    )(cat3, rot3, dl, Wc_p, Wo_p, Wn1, Wn2, Wn3, Wn4,
      W_size, W_trans, bs2, bt2, bn2)

    return (m_node, m_edge)
